# Initial kernel scaffold; baseline (speedup 1.0000x reference)
import jax
import jax.numpy as jnp
from jax.experimental import pallas as pl


def kernel(flux, tile_idx, fib_idx, eta):
    raise NotImplementedError("write your pallas kernel here")


# speedup vs baseline: 6.9536x; 6.9536x over previous
import jax
import jax.numpy as jnp
from jax import lax
from jax.experimental import pallas as pl
from jax.experimental.pallas import tpu as pltpu
from jax.experimental.pallas import tpu_sc as plsc

N_TILES = 8
N_FIBRES = 5000
J = 20000
JP = 20480
B = 128
BC = 32
NCHUNK = B // BC
ROWS_PAD = N_FIBRES + 2
A_ROWS = 40960
PER_TILE = JP // 16
HALF = PER_TILE // 2
IDXR = PER_TILE // 128


def _body(flux2, tid, fid, etav, out2, A, F, G0, G1, G2, Z,
          tI, fI, c0, cm, cp, ev, sem, sem2):
    cid = lax.axis_index("c")
    sid = lax.axis_index("s")

    pltpu.sync_copy(etav, ev)
    e = ev[...]
    cc = 1.0 - 2.0 * e
    cn = e

    ibase = pl.multiple_of(sid * PER_TILE, 8)
    pltpu.sync_copy(tid.at[pl.ds(ibase, PER_TILE)], tI)
    pltpu.sync_copy(fid.at[pl.ds(ibase, PER_TILE)], fI)
    for r in range(IDXR):
        for l in range(8):
            s = pl.ds(l * 16, 16)
            e0 = pl.ds(r * 128 + l * 16, 16)
            c = tI[e0] * ROWS_PAD + fI[e0] + 1
            c0[r, s] = c
            cm[r, s] = c - 1
            cp[r, s] = c + 1

    z16 = jnp.zeros((16,), jnp.float32)

    def _zb(i, carry):
        Z[i, pl.ds(0, 16)] = z16
        Z[i, pl.ds(16, 16)] = z16
        return carry

    lax.fori_loop(0, 128, _zb, 0)

    for k in range(2):
        ci = cid * 2 + k
        zd = [pltpu.async_copy(Z, A.at[pl.ds(sid * 2560 + z * 128, 128)], sem)
              for z in range(2560 // 128)]
        for d in zd:
            d.wait()
        plsc.subcore_barrier()

        base = pl.multiple_of(ci * JP + sid * PER_TILE, 8)
        for s in range(IDXR):
            pltpu.sync_copy(flux2.at[pl.ds(base + s * 128, 128)], F)
            pltpu.sync_copy(F, A.at[c0.at[s]], add=True)
        plsc.subcore_barrier()

        for s in range(IDXR):
            gd = [pltpu.async_copy(A.at[c0.at[s]], G0, sem),
                  pltpu.async_copy(A.at[cm.at[s]], G1, sem),
                  pltpu.async_copy(A.at[cp.at[s]], G2, sem)]
            for d in gd:
                d.wait()

            def _comb(i, carry):
                r0 = i * 4
                for u in range(4):
                    for c2 in (0, 16):
                        sl = pl.ds(c2, 16)
                        g0 = G0[r0 + u, sl]
                        g12 = G1[r0 + u, sl] + G2[r0 + u, sl]
                        G0[r0 + u, sl] = g0 * cc + g12 * cn
                return carry

            lax.fori_loop(0, 128 // 4, _comb, 0)
            pltpu.sync_copy(G0, out2.at[pl.ds(base + s * 128, 128)])
        plsc.subcore_barrier()


def _sc_call(flux2, tid, fid, etav):
    mesh = plsc.VectorSubcoreMesh(core_axis_name="c", subcore_axis_name="s")
    return pl.kernel(
        _body,
        out_type=jax.ShapeDtypeStruct((NCHUNK * JP, BC), jnp.float32),
        mesh=mesh,
        compiler_params=pltpu.CompilerParams(use_tc_tiling_on_sc=False),
        scratch_types=[
            pltpu.VMEM_SHARED((A_ROWS, BC), jnp.float32),
            pltpu.VMEM((128, BC), jnp.float32),
            pltpu.VMEM((128, BC), jnp.float32),
            pltpu.VMEM((128, BC), jnp.float32),
            pltpu.VMEM((128, BC), jnp.float32),
            pltpu.VMEM((128, BC), jnp.float32),
            pltpu.VMEM((PER_TILE,), jnp.int32),
            pltpu.VMEM((PER_TILE,), jnp.int32),
            pltpu.VMEM((IDXR, 128), jnp.int32),
            pltpu.VMEM((IDXR, 128), jnp.int32),
            pltpu.VMEM((IDXR, 128), jnp.int32),
            pltpu.VMEM((16,), jnp.float32),
            pltpu.SemaphoreType.DMA,
            pltpu.SemaphoreType.DMA,
        ],
    )(flux2, tid, fid, etav)


def kernel(flux, tile_idx, fib_idx, eta):
    flux_p = jnp.pad(flux, ((0, 0), (0, JP - J)))
    flux2 = (flux_p.reshape(NCHUNK, BC, JP).transpose(0, 2, 1)
             .reshape(NCHUNK * JP, BC))
    tid = jnp.pad(tile_idx, (0, JP - J))
    fid = jnp.pad(fib_idx, (0, JP - J))
    etav = jnp.full((16,), eta, jnp.float32)
    out2 = _sc_call(flux2, tid, fid, etav)
    out = (out2.reshape(NCHUNK, JP, BC)[:, :J].transpose(0, 2, 1)
           .reshape(B, J))
    return out


# speedup vs baseline: 7.8546x; 1.1296x over previous
import jax
import jax.numpy as jnp
from jax import lax
from jax.experimental import pallas as pl
from jax.experimental.pallas import tpu as pltpu
from jax.experimental.pallas import tpu_sc as plsc

N_TILES = 8
N_FIBRES = 5000
J = 20000
JP = 20480
B = 128
BC = 32
NCHUNK = B // BC
ROWS_PAD = N_FIBRES + 2
A_ROWS = 40960
PER_TILE = JP // 16
HALF = PER_TILE // 2
IDXR = PER_TILE // 128


def _body(flux2, tid, fid, etav, out2, A, F, G0, G1, G2, Z,
          tI, fI, c0, cm, cp, ev, sem, sem2):
    cid = lax.axis_index("c")
    sid = lax.axis_index("s")

    pltpu.sync_copy(etav, ev)
    e = ev[...]
    cc = 1.0 - 2.0 * e
    cn = e

    ibase = pl.multiple_of(sid * PER_TILE, 8)
    pltpu.sync_copy(tid.at[pl.ds(ibase, PER_TILE)], tI)
    pltpu.sync_copy(fid.at[pl.ds(ibase, PER_TILE)], fI)
    for r in range(IDXR):
        for l in range(8):
            s = pl.ds(l * 16, 16)
            e0 = pl.ds(r * 128 + l * 16, 16)
            c = tI[e0] * ROWS_PAD + fI[e0] + 1
            c0[r, s] = c
            cm[r, s] = c - 1
            cp[r, s] = c + 1

    z16 = jnp.zeros((16,), jnp.float32)

    def _zb(i, carry):
        Z[i, pl.ds(0, 16)] = z16
        Z[i, pl.ds(16, 16)] = z16
        return carry

    lax.fori_loop(0, 128, _zb, 0)

    for k in range(2):
        ci = cid * 2 + k
        zd = [pltpu.async_copy(Z, A.at[pl.ds(sid * 2560 + z * 128, 128)], sem)
              for z in range(2560 // 128)]
        for d in zd:
            d.wait()
        plsc.subcore_barrier()

        base = pl.multiple_of(ci * JP + sid * PER_TILE, 8)
        for s in range(IDXR):
            pltpu.sync_copy(flux2.at[pl.ds(base + s * 128, 128)], F)
            pltpu.sync_copy(F, A.at[c0.at[s]], add=True)
        plsc.subcore_barrier()

        for s in range(IDXR):
            gd = [pltpu.async_copy(A.at[c0.at[s]], G0, sem),
                  pltpu.async_copy(A.at[cm.at[s]], G1, sem),
                  pltpu.async_copy(A.at[cp.at[s]], G2, sem)]
            for d in gd:
                d.wait()

            def _comb(i, carry):
                r0 = i * 4
                for u in range(4):
                    for c2 in (0, 16):
                        sl = pl.ds(c2, 16)
                        g0 = G0[r0 + u, sl]
                        g12 = G1[r0 + u, sl] + G2[r0 + u, sl]
                        G0[r0 + u, sl] = g0 * cc + g12 * cn
                return carry

            lax.fori_loop(0, 128 // 4, _comb, 0)
            pltpu.sync_copy(G0, out2.at[pl.ds(base + s * 128, 128)])
        plsc.subcore_barrier()


def _sc_call(flux2, tid, fid, etav):
    mesh = plsc.VectorSubcoreMesh(core_axis_name="c", subcore_axis_name="s")
    return pl.kernel(
        _body,
        out_type=jax.ShapeDtypeStruct((NCHUNK * JP, BC), jnp.float32),
        mesh=mesh,
        compiler_params=pltpu.CompilerParams(use_tc_tiling_on_sc=False),
        scratch_types=[
            pltpu.VMEM_SHARED((A_ROWS, BC), jnp.float32),
            pltpu.VMEM((128, BC), jnp.float32),
            pltpu.VMEM((128, BC), jnp.float32),
            pltpu.VMEM((128, BC), jnp.float32),
            pltpu.VMEM((128, BC), jnp.float32),
            pltpu.VMEM((128, BC), jnp.float32),
            pltpu.VMEM((PER_TILE,), jnp.int32),
            pltpu.VMEM((PER_TILE,), jnp.int32),
            pltpu.VMEM((IDXR, 128), jnp.int32),
            pltpu.VMEM((IDXR, 128), jnp.int32),
            pltpu.VMEM((IDXR, 128), jnp.int32),
            pltpu.VMEM((16,), jnp.float32),
            pltpu.SemaphoreType.DMA,
            pltpu.SemaphoreType.DMA,
        ],
    )(flux2, tid, fid, etav)


def _tin_body(fx_ref, o_ref):
    o_ref[pl.ds(0, J), :] = fx_ref[...].T


def _transpose_in(flux):
    return pl.pallas_call(
        _tin_body,
        grid=(NCHUNK,),
        in_specs=[pl.BlockSpec((BC, J), lambda c: (c, 0))],
        out_specs=pl.BlockSpec((JP, BC), lambda c: (c, 0)),
        out_shape=jax.ShapeDtypeStruct((NCHUNK * JP, BC), jnp.float32),
    )(flux)


def _tout_body(o2_ref, o_ref):
    o_ref[...] = o2_ref[pl.ds(0, J), :].T


def _transpose_out(out2):
    return pl.pallas_call(
        _tout_body,
        grid=(NCHUNK,),
        in_specs=[pl.BlockSpec((JP, BC), lambda c: (c, 0))],
        out_specs=pl.BlockSpec((BC, J), lambda c: (c, 0)),
        out_shape=jax.ShapeDtypeStruct((B, J), jnp.float32),
    )(out2)


def kernel(flux, tile_idx, fib_idx, eta):
    flux2 = _transpose_in(flux)
    tid = jnp.pad(tile_idx, (0, JP - J), constant_values=8)
    fid = jnp.pad(fib_idx, (0, JP - J), constant_values=483)
    etav = jnp.full((16,), eta, jnp.float32)
    out2 = _sc_call(flux2, tid, fid, etav)
    return _transpose_out(out2)


# speedup vs baseline: 12.9832x; 1.6529x over previous
import jax
import jax.numpy as jnp
from jax import lax
from jax.experimental import pallas as pl
from jax.experimental.pallas import tpu as pltpu
from jax.experimental.pallas import tpu_sc as plsc

N_TILES = 8
N_FIBRES = 5000
J = 20000
JP = 20480
B = 128
BC = 32
NCHUNK = B // BC
ROWS_PAD = N_FIBRES + 2
A_ROWS = 40960
PER_TILE = JP // 16
HALF = PER_TILE // 2
IDXR = PER_TILE // 128


def _body(flux2, tid, fid, etav, out2, A, F, G0, G1, G2, Z,
          tI, fI, c0, cm, cp, ev, sem, sem2):
    cid = lax.axis_index("c")
    sid = lax.axis_index("s")

    pltpu.sync_copy(etav, ev)
    e = ev[...]
    cc = 1.0 - 2.0 * e
    cn = e

    ibase = pl.multiple_of(sid * PER_TILE, 8)
    pltpu.sync_copy(tid.at[pl.ds(ibase, PER_TILE)], tI)
    pltpu.sync_copy(fid.at[pl.ds(ibase, PER_TILE)], fI)
    for r in range(IDXR):
        for l in range(8):
            s = pl.ds(l * 16, 16)
            e0 = pl.ds(r * 128 + l * 16, 16)
            c = tI[e0] * ROWS_PAD + fI[e0] + 1
            c0[r, s] = c
            cm[r, s] = c - 1
            cp[r, s] = c + 1

    z16 = jnp.zeros((16,), jnp.float32)

    def _zb(i, carry):
        Z[i, pl.ds(0, 16)] = z16
        Z[i, pl.ds(16, 16)] = z16
        return carry

    lax.fori_loop(0, 128, _zb, 0)

    jbase = sid * PER_TILE
    for k in range(2):
        ci = cid * 2 + k
        co = pl.multiple_of(ci * BC, 8)
        zd = [pltpu.async_copy(Z, A.at[pl.ds(sid * 2560 + z * 128, 128)], sem)
              for z in range(2560 // 128)]
        for d in zd:
            d.wait()
        plsc.subcore_barrier()

        for s in range(IDXR):
            pltpu.sync_copy(
                flux2.at[pl.ds(jbase + s * 128, 128), pl.ds(co, BC)], F)
            pltpu.sync_copy(F, A.at[c0.at[s]], add=True)
        plsc.subcore_barrier()

        for s in range(IDXR):
            gd = [pltpu.async_copy(A.at[c0.at[s]], G0, sem),
                  pltpu.async_copy(A.at[cm.at[s]], G1, sem),
                  pltpu.async_copy(A.at[cp.at[s]], G2, sem)]
            for d in gd:
                d.wait()

            def _comb(i, carry):
                r0 = i * 4
                for u in range(4):
                    for c2 in (0, 16):
                        sl = pl.ds(c2, 16)
                        g0 = G0[r0 + u, sl]
                        g12 = G1[r0 + u, sl] + G2[r0 + u, sl]
                        G0[r0 + u, sl] = g0 * cc + g12 * cn
                return carry

            lax.fori_loop(0, 128 // 4, _comb, 0)
            pltpu.sync_copy(
                G0, out2.at[pl.ds(jbase + s * 128, 128), pl.ds(co, BC)])
        plsc.subcore_barrier()


def _sc_call(flux2, tid, fid, etav):
    mesh = plsc.VectorSubcoreMesh(core_axis_name="c", subcore_axis_name="s")
    return pl.kernel(
        _body,
        out_type=jax.ShapeDtypeStruct((JP, B), jnp.float32),
        mesh=mesh,
        compiler_params=pltpu.CompilerParams(use_tc_tiling_on_sc=False),
        scratch_types=[
            pltpu.VMEM_SHARED((A_ROWS, BC), jnp.float32),
            pltpu.VMEM((128, BC), jnp.float32),
            pltpu.VMEM((128, BC), jnp.float32),
            pltpu.VMEM((128, BC), jnp.float32),
            pltpu.VMEM((128, BC), jnp.float32),
            pltpu.VMEM((128, BC), jnp.float32),
            pltpu.VMEM((PER_TILE,), jnp.int32),
            pltpu.VMEM((PER_TILE,), jnp.int32),
            pltpu.VMEM((IDXR, 128), jnp.int32),
            pltpu.VMEM((IDXR, 128), jnp.int32),
            pltpu.VMEM((IDXR, 128), jnp.int32),
            pltpu.VMEM((16,), jnp.float32),
            pltpu.SemaphoreType.DMA,
            pltpu.SemaphoreType.DMA,
        ],
    )(flux2, tid, fid, etav)


JB = 4096


def _tin_body(fx_ref, o_ref):
    o_ref[...] = fx_ref[...].T


def _transpose_in(flux):
    return pl.pallas_call(
        _tin_body,
        grid=(JP // JB,),
        in_specs=[pl.BlockSpec((B, JB), lambda j: (0, j))],
        out_specs=pl.BlockSpec((JB, B), lambda j: (j, 0)),
        out_shape=jax.ShapeDtypeStruct((JP, B), jnp.float32),
    )(flux)


def _tout_body(o2_ref, o_ref):
    o_ref[...] = o2_ref[...].T


def _transpose_out(out2):
    return pl.pallas_call(
        _tout_body,
        grid=(JP // JB,),
        in_specs=[pl.BlockSpec((JB, B), lambda j: (j, 0))],
        out_specs=pl.BlockSpec((B, JB), lambda j: (0, j)),
        out_shape=jax.ShapeDtypeStruct((B, J), jnp.float32),
    )(out2)


def kernel(flux, tile_idx, fib_idx, eta):
    flux2 = _transpose_in(flux)
    tid = jnp.pad(tile_idx, (0, JP - J), constant_values=8)
    fid = jnp.pad(fib_idx, (0, JP - J), constant_values=483)
    etav = jnp.full((16,), eta, jnp.float32)
    out2 = _sc_call(flux2, tid, fid, etav)
    return _transpose_out(out2)


# speedup vs baseline: 19.0833x; 1.4699x over previous
import jax
import jax.numpy as jnp
from jax import lax
from jax.experimental import pallas as pl
from jax.experimental.pallas import tpu as pltpu
from jax.experimental.pallas import tpu_sc as plsc

N_TILES = 8
N_FIBRES = 5000
J = 20000
B = 128
BC = 32
NCHUNK = B // BC
ROWS_PAD = N_FIBRES + 2
A_ROWS = 40960
PER_TILE = 1280
IDXR = PER_TILE // 128
LAST = J - 15 * PER_TILE
DUMP_T = 8
DUMP_F = 483


def _body(flux2, tid, fid, etav, out2, A, F, G0, G1, G2, Z,
          tI, fI, c0, cm, cp, ev, sem, sem2):
    cid = lax.axis_index("c")
    sid = lax.axis_index("s")

    pltpu.sync_copy(etav, ev)
    e = ev[...]
    cc = 1.0 - 2.0 * e
    cn = e

    jbase = sid * PER_TILE

    @pl.when(sid < 15)
    def _():
        pltpu.sync_copy(tid.at[pl.ds(jbase, PER_TILE)], tI)
        pltpu.sync_copy(fid.at[pl.ds(jbase, PER_TILE)], fI)

    @pl.when(sid == 15)
    def _():
        pltpu.sync_copy(tid.at[pl.ds(15 * PER_TILE, LAST)],
                        tI.at[pl.ds(0, LAST)])
        pltpu.sync_copy(fid.at[pl.ds(15 * PER_TILE, LAST)],
                        fI.at[pl.ds(0, LAST)])
        dt = jnp.full((16,), DUMP_T, jnp.int32)
        df = jnp.full((16,), DUMP_F, jnp.int32)
        for q in range(LAST, PER_TILE, 16):
            tI[pl.ds(q, 16)] = dt
            fI[pl.ds(q, 16)] = df

    for r in range(IDXR):
        for l in range(8):
            s = pl.ds(l * 16, 16)
            e0 = pl.ds(r * 128 + l * 16, 16)
            c = tI[e0] * ROWS_PAD + fI[e0] + 1
            c0[r, s] = c
            cm[r, s] = c - 1
            cp[r, s] = c + 1

    z16 = jnp.zeros((16,), jnp.float32)

    def _zb(i, carry):
        Z[i, pl.ds(0, 16)] = z16
        Z[i, pl.ds(16, 16)] = z16
        return carry

    lax.fori_loop(0, 128, _zb, 0)

    def _comb(i, carry):
        r0 = i * 4
        for u in range(4):
            for c2 in (0, 16):
                sl = pl.ds(c2, 16)
                g0 = G0[r0 + u, sl]
                g12 = G1[r0 + u, sl] + G2[r0 + u, sl]
                G0[r0 + u, sl] = g0 * cc + g12 * cn
        return carry

    def _load_scatter(s, co, rows):
        pltpu.sync_copy(
            flux2.at[pl.ds(jbase + s * 128, rows), pl.ds(co, BC)],
            F if rows == 128 else F.at[pl.ds(0, rows)])
        pltpu.sync_copy(F, A.at[c0.at[s]], add=True)

    def _gather_combine_store(s, co, rows):
        gd = [pltpu.async_copy(A.at[c0.at[s]], G0, sem),
              pltpu.async_copy(A.at[cm.at[s]], G1, sem),
              pltpu.async_copy(A.at[cp.at[s]], G2, sem)]
        for d in gd:
            d.wait()
        lax.fori_loop(0, 128 // 4, _comb, 0)
        pltpu.sync_copy(
            G0 if rows == 128 else G0.at[pl.ds(0, rows)],
            out2.at[pl.ds(jbase + s * 128, rows), pl.ds(co, BC)])

    for k in range(2):
        ci = cid * 2 + k
        co = pl.multiple_of(ci * BC, 8)
        zd = [pltpu.async_copy(Z, A.at[pl.ds(sid * 2560 + z * 128, 128)], sem)
              for z in range(2560 // 128)]
        for d in zd:
            d.wait()
        plsc.subcore_barrier()

        @pl.when(sid < 15)
        def _():
            for s in range(IDXR):
                _load_scatter(s, co, 128)

        @pl.when(sid == 15)
        def _():
            for s in range(LAST // 128):
                _load_scatter(s, co, 128)
            _load_scatter(LAST // 128, co, LAST % 128)
        plsc.subcore_barrier()

        @pl.when(sid < 15)
        def _():
            for s in range(IDXR):
                _gather_combine_store(s, co, 128)

        @pl.when(sid == 15)
        def _():
            for s in range(LAST // 128):
                _gather_combine_store(s, co, 128)
            _gather_combine_store(LAST // 128, co, LAST % 128)
        plsc.subcore_barrier()


def _sc_call(flux2, tid, fid, etav):
    mesh = plsc.VectorSubcoreMesh(core_axis_name="c", subcore_axis_name="s")
    return pl.kernel(
        _body,
        out_type=jax.ShapeDtypeStruct((J, B), jnp.float32),
        mesh=mesh,
        compiler_params=pltpu.CompilerParams(use_tc_tiling_on_sc=False),
        scratch_types=[
            pltpu.VMEM_SHARED((A_ROWS, BC), jnp.float32),
            pltpu.VMEM((128, BC), jnp.float32),
            pltpu.VMEM((128, BC), jnp.float32),
            pltpu.VMEM((128, BC), jnp.float32),
            pltpu.VMEM((128, BC), jnp.float32),
            pltpu.VMEM((128, BC), jnp.float32),
            pltpu.VMEM((PER_TILE,), jnp.int32),
            pltpu.VMEM((PER_TILE,), jnp.int32),
            pltpu.VMEM((IDXR, 128), jnp.int32),
            pltpu.VMEM((IDXR, 128), jnp.int32),
            pltpu.VMEM((IDXR, 128), jnp.int32),
            pltpu.VMEM((16,), jnp.float32),
            pltpu.SemaphoreType.DMA,
            pltpu.SemaphoreType.DMA,
        ],
    )(flux2, tid, fid, etav)


def kernel(flux, tile_idx, fib_idx, eta):
    etav = jnp.full((16,), eta, jnp.float32)
    out2 = _sc_call(flux.T, tile_idx, fib_idx, etav)
    return out2.T


# speedup vs baseline: 24.8921x; 1.3044x over previous
import jax
import jax.numpy as jnp
from jax import lax
from jax.experimental import pallas as pl
from jax.experimental.pallas import tpu as pltpu
from jax.experimental.pallas import tpu_sc as plsc

N_TILES = 8
N_FIBRES = 5000
J = 20000
B = 128
BC = 32
NCHUNK = B // BC
ROWS_PAD = N_FIBRES + 2
A_ROWS = 40960
PER_TILE = 1280
IDXR = PER_TILE // 128
LAST = J - 15 * PER_TILE
WAVE = 4
DUMP_T = 8
DUMP_F = 483


def _body(flux2, tid, fid, etav, out2,
          A, B0, B1, B2, B3, B4, B5, B6, B7, Z,
          tI, fI, c0, cm, cp, ev,
          semL, semS0, semS1, semG0, semG1, semO0, semO1):
    cid = lax.axis_index("c")
    sid = lax.axis_index("s")
    Bufs = (B0, B1, B2, B3, B4, B5, B6, B7)
    semS = (semS0, semS1)
    semG = (semG0, semG1)
    semO = (semO0, semO1)
    Obuf = (B6, B7)

    pltpu.sync_copy(etav, ev)
    e = ev[...]
    cc = 1.0 - 2.0 * e
    cn = e

    jbase = sid * PER_TILE

    @pl.when(sid < 15)
    def _():
        pltpu.sync_copy(tid.at[pl.ds(jbase, PER_TILE)], tI)
        pltpu.sync_copy(fid.at[pl.ds(jbase, PER_TILE)], fI)

    @pl.when(sid == 15)
    def _():
        pltpu.sync_copy(tid.at[pl.ds(15 * PER_TILE, LAST)],
                        tI.at[pl.ds(0, LAST)])
        pltpu.sync_copy(fid.at[pl.ds(15 * PER_TILE, LAST)],
                        fI.at[pl.ds(0, LAST)])
        dt = jnp.full((16,), DUMP_T, jnp.int32)
        df = jnp.full((16,), DUMP_F, jnp.int32)
        for q in range(LAST, PER_TILE, 16):
            tI[pl.ds(q, 16)] = dt
            fI[pl.ds(q, 16)] = df

    for r in range(IDXR):
        for l in range(8):
            s = pl.ds(l * 16, 16)
            e0 = pl.ds(r * 128 + l * 16, 16)
            c = tI[e0] * ROWS_PAD + fI[e0] + 1
            c0[r, s] = c
            cm[r, s] = c - 1
            cp[r, s] = c + 1

    z16 = jnp.zeros((16,), jnp.float32)

    def _zb(i, carry):
        Z[i, pl.ds(0, 16)] = z16
        Z[i, pl.ds(16, 16)] = z16
        return carry

    lax.fori_loop(0, 128, _zb, 0)

    def _scatter_phase(pieces, co):
        waves = [pieces[i:i + WAVE] for i in range(0, len(pieces), WAVE)]

        def _load(s, rows, b):
            src = flux2.at[pl.ds(jbase + s * 128, rows), pl.ds(co, BC)]
            dst = Bufs[b] if rows == 128 else Bufs[b].at[pl.ds(0, rows)]
            return pltpu.async_copy(src, dst, semL)

        def _fire_loads(w):
            g = (w % 2) * WAVE
            return [_load(s, rows, g + i)
                    for i, (s, rows) in enumerate(waves[w])]

        dl = _fire_loads(0)
        dsc = {}
        for w in range(len(waves)):
            for d in dl:
                d.wait()
            g = (w % 2) * WAVE
            dsc[w] = [pltpu.async_copy(Bufs[g + i], A.at[c0.at[s]],
                                       semS[w % 2], add=True)
                      for i, (s, _) in enumerate(waves[w])]
            if w + 1 < len(waves):
                if w - 1 >= 0:
                    for d in dsc[w - 1]:
                        d.wait()
                dl = _fire_loads(w + 1)
        for w in range(max(0, len(waves) - 2), len(waves)):
            for d in dsc[w]:
                d.wait()

    def _comb(t, dst):
        g0r, g1r, g2r = Bufs[t * 3], Bufs[t * 3 + 1], Bufs[t * 3 + 2]
        O = Obuf[dst]

        def body(i, carry):
            r0 = i * 4
            for u in range(4):
                for c2 in (0, 16):
                    sl = pl.ds(c2, 16)
                    g0 = g0r[r0 + u, sl]
                    g12 = g1r[r0 + u, sl] + g2r[r0 + u, sl]
                    O[r0 + u, sl] = g0 * cc + g12 * cn
            return carry

        lax.fori_loop(0, 128 // 4, body, 0)

    def _gather_phase(pieces, co):
        npc = len(pieces)

        def _fire_g(s):
            t = (s % 2) * 3
            sg = semG[s % 2]
            return [pltpu.async_copy(A.at[c0.at[s]], Bufs[t], sg),
                    pltpu.async_copy(A.at[cm.at[s]], Bufs[t + 1], sg),
                    pltpu.async_copy(A.at[cp.at[s]], Bufs[t + 2], sg)]

        gd = {0: _fire_g(pieces[0][0])}
        if npc > 1:
            gd[1] = _fire_g(pieces[1][0])
        od = {}
        for i in range(npc):
            s, rows = pieces[i]
            for d in gd[i]:
                d.wait()
            if i >= 2:
                od[i - 2].wait()
            _comb(i % 2, i % 2)
            if i + 2 < npc:
                gd[i + 2] = _fire_g(pieces[i + 2][0])
            O = Obuf[i % 2]
            src = O if rows == 128 else O.at[pl.ds(0, rows)]
            od[i] = pltpu.async_copy(
                src, out2.at[pl.ds(jbase + s * 128, rows), pl.ds(co, BC)],
                semO[i % 2])
        for i in range(max(0, npc - 2), npc):
            od[i].wait()

    full = [(s, 128) for s in range(IDXR)]
    short = [(s, 128) for s in range(LAST // 128)] + [(LAST // 128, LAST % 128)]

    for k in range(2):
        ci = cid * 2 + k
        co = pl.multiple_of(ci * BC, 8)
        zd = [pltpu.async_copy(Z, A.at[pl.ds(sid * 2560 + z * 128, 128)],
                               semL)
              for z in range(2560 // 128)]
        for d in zd:
            d.wait()
        plsc.subcore_barrier()

        @pl.when(sid < 15)
        def _():
            _scatter_phase(full, co)

        @pl.when(sid == 15)
        def _():
            _scatter_phase(short, co)
        plsc.subcore_barrier()

        @pl.when(sid < 15)
        def _():
            _gather_phase(full, co)

        @pl.when(sid == 15)
        def _():
            _gather_phase(short, co)
        plsc.subcore_barrier()


def _sc_call(flux2, tid, fid, etav):
    mesh = plsc.VectorSubcoreMesh(core_axis_name="c", subcore_axis_name="s")
    return pl.kernel(
        _body,
        out_type=jax.ShapeDtypeStruct((J, B), jnp.float32),
        mesh=mesh,
        compiler_params=pltpu.CompilerParams(use_tc_tiling_on_sc=False),
        scratch_types=[
            pltpu.VMEM_SHARED((A_ROWS, BC), jnp.float32),
        ] + [pltpu.VMEM((128, BC), jnp.float32)] * 9 +
        [
            pltpu.VMEM((PER_TILE,), jnp.int32),
            pltpu.VMEM((PER_TILE,), jnp.int32),
            pltpu.VMEM((IDXR, 128), jnp.int32),
            pltpu.VMEM((IDXR, 128), jnp.int32),
            pltpu.VMEM((IDXR, 128), jnp.int32),
            pltpu.VMEM((16,), jnp.float32),
        ] + [pltpu.SemaphoreType.DMA] * 7,
    )(flux2, tid, fid, etav)


def kernel(flux, tile_idx, fib_idx, eta):
    etav = jnp.full((16,), eta, jnp.float32)
    out2 = _sc_call(flux.T, tile_idx, fib_idx, etav)
    return out2.T


# speedup vs baseline: 26.5927x; 1.0683x over previous
import jax
import jax.numpy as jnp
from jax import lax
from jax.experimental import pallas as pl
from jax.experimental.pallas import tpu as pltpu
from jax.experimental.pallas import tpu_sc as plsc

N_TILES = 8
N_FIBRES = 5000
J = 20000
B = 128
BC = 32
NCHUNK = B // BC
ROWS_PAD = N_FIBRES + 2
A_ROWS = 40960
PER_TILE = 1280
IDXR = PER_TILE // 128
LAST = J - 15 * PER_TILE
WAVE = 4
DUMP_T = 8
DUMP_F = 483


def _body(flux2, tid, fid, etav, out2,
          A, B0, B1, B2, B3, B4, B5, B6, B7, Z,
          tI, fI, c0, cm, cp, ev,
          semL, semS0, semS1, semG0, semG1, semO0, semO1):
    cid = lax.axis_index("c")
    sid = lax.axis_index("s")
    Bufs = (B0, B1, B2, B3, B4, B5, B6, B7)
    semS = (semS0, semS1)
    semG = (semG0, semG1)
    semO = (semO0, semO1)
    Obuf = (B6, B7)

    pltpu.sync_copy(etav, ev)
    e = ev[...]
    cc = 1.0 - 2.0 * e
    cn = e

    jbase = sid * PER_TILE

    @pl.when(sid < 15)
    def _():
        pltpu.async_copy(tid.at[pl.ds(jbase, PER_TILE)], tI, semG0).wait()
        pltpu.async_copy(fid.at[pl.ds(jbase, PER_TILE)], fI, semG1).wait()

    @pl.when(sid == 15)
    def _():
        pltpu.async_copy(tid.at[pl.ds(15 * PER_TILE, LAST)],
                         tI.at[pl.ds(0, LAST)], semG0).wait()
        pltpu.async_copy(fid.at[pl.ds(15 * PER_TILE, LAST)],
                         fI.at[pl.ds(0, LAST)], semG1).wait()
        dt = jnp.full((16,), DUMP_T, jnp.int32)
        df = jnp.full((16,), DUMP_F, jnp.int32)
        for q in range(LAST, PER_TILE, 16):
            tI[pl.ds(q, 16)] = dt
            fI[pl.ds(q, 16)] = df

    def _compute_indices():
        for r in range(IDXR):
            for l in range(8):
                s = pl.ds(l * 16, 16)
                e0 = pl.ds(r * 128 + l * 16, 16)
                c = tI[e0] * ROWS_PAD + fI[e0] + 1
                c0[r, s] = c
                cm[r, s] = c - 1
                cp[r, s] = c + 1

    z16 = jnp.zeros((16,), jnp.float32)

    def _zb(i, carry):
        Z[i, pl.ds(0, 16)] = z16
        Z[i, pl.ds(16, 16)] = z16
        return carry

    def _fire_zero():
        return [pltpu.async_copy(
                    Z, A.at[pl.ds(sid * 2560 + z * 256, 256)], semL)
                for z in range(2560 // 256)]

    def _scatter_phase(pieces, co):
        waves = [pieces[i:i + WAVE] for i in range(0, len(pieces), WAVE)]

        def _load(s, rows, b):
            src = flux2.at[pl.ds(jbase + s * 128, rows), pl.ds(co, BC)]
            dst = Bufs[b] if rows == 128 else Bufs[b].at[pl.ds(0, rows)]
            return pltpu.async_copy(src, dst, semL)

        def _fire_loads(w):
            g = (w % 2) * WAVE
            return [_load(s, rows, g + i)
                    for i, (s, rows) in enumerate(waves[w])]

        dl = _fire_loads(0)
        dsc = {}
        for w in range(len(waves)):
            for d in dl:
                d.wait()
            g = (w % 2) * WAVE
            dsc[w] = [pltpu.async_copy(Bufs[g + i], A.at[c0.at[s]],
                                       semS[w % 2], add=True)
                      for i, (s, _) in enumerate(waves[w])]
            if w + 1 < len(waves):
                if w - 1 >= 0:
                    for d in dsc[w - 1]:
                        d.wait()
                dl = _fire_loads(w + 1)
        for w in range(max(0, len(waves) - 2), len(waves)):
            for d in dsc[w]:
                d.wait()

    def _comb(t, dst):
        g0r, g1r, g2r = Bufs[t * 3], Bufs[t * 3 + 1], Bufs[t * 3 + 2]
        O = Obuf[dst]

        def body(i, carry):
            r0 = i * 4
            for u in range(4):
                for c2 in (0, 16):
                    sl = pl.ds(c2, 16)
                    g0 = g0r[r0 + u, sl]
                    g12 = g1r[r0 + u, sl] + g2r[r0 + u, sl]
                    O[r0 + u, sl] = g0 * cc + g12 * cn
            return carry

        lax.fori_loop(0, 128 // 4, body, 0)

    def _gather_phase(pieces, co):
        npc = len(pieces)

        def _fire_g(s):
            t = (s % 2) * 3
            sg = semG[s % 2]
            return [pltpu.async_copy(A.at[c0.at[s]], Bufs[t], sg),
                    pltpu.async_copy(A.at[cm.at[s]], Bufs[t + 1], sg),
                    pltpu.async_copy(A.at[cp.at[s]], Bufs[t + 2], sg)]

        gd = {0: _fire_g(pieces[0][0])}
        if npc > 1:
            gd[1] = _fire_g(pieces[1][0])
        od = {}
        for i in range(npc):
            s, rows = pieces[i]
            for d in gd[i]:
                d.wait()
            if i >= 2:
                od[i - 2].wait()
            _comb(i % 2, i % 2)
            if i + 2 < npc:
                gd[i + 2] = _fire_g(pieces[i + 2][0])
            O = Obuf[i % 2]
            src = O if rows == 128 else O.at[pl.ds(0, rows)]
            od[i] = pltpu.async_copy(
                src, out2.at[pl.ds(jbase + s * 128, rows), pl.ds(co, BC)],
                semO[i % 2])
        for i in range(max(0, npc - 2), npc):
            od[i].wait()

    full = [(s, 128) for s in range(IDXR)]
    short = [(s, 128) for s in range(LAST // 128)] + [(LAST // 128, LAST % 128)]

    lax.fori_loop(0, 256, _zb, 0)
    zd = _fire_zero()
    _compute_indices()
    for d in zd:
        d.wait()
    plsc.subcore_barrier()

    def _chunk(k, carry):
        ci = cid * 2 + k
        co = pl.multiple_of(ci * BC, 8)

        @pl.when(sid < 15)
        def _():
            _scatter_phase(full, co)

        @pl.when(sid == 15)
        def _():
            _scatter_phase(short, co)
        plsc.subcore_barrier()

        @pl.when(sid < 15)
        def _():
            _gather_phase(full, co)

        @pl.when(sid == 15)
        def _():
            _gather_phase(short, co)
        plsc.subcore_barrier()

        @pl.when(k == 0)
        def _():
            zd2 = _fire_zero()
            for d in zd2:
                d.wait()
            plsc.subcore_barrier()
        return carry

    lax.fori_loop(0, 2, _chunk, 0)


def _sc_call(flux2, tid, fid, etav):
    mesh = plsc.VectorSubcoreMesh(core_axis_name="c", subcore_axis_name="s")
    return pl.kernel(
        _body,
        out_type=jax.ShapeDtypeStruct((J, B), jnp.float32),
        mesh=mesh,
        compiler_params=pltpu.CompilerParams(use_tc_tiling_on_sc=False),
        scratch_types=[
            pltpu.VMEM_SHARED((A_ROWS, BC), jnp.float32),
        ] + [pltpu.VMEM((128, BC), jnp.float32)] * 8 +
        [
            pltpu.VMEM((256, BC), jnp.float32),
            pltpu.VMEM((PER_TILE,), jnp.int32),
            pltpu.VMEM((PER_TILE,), jnp.int32),
            pltpu.VMEM((IDXR, 128), jnp.int32),
            pltpu.VMEM((IDXR, 128), jnp.int32),
            pltpu.VMEM((IDXR, 128), jnp.int32),
            pltpu.VMEM((16,), jnp.float32),
        ] + [pltpu.SemaphoreType.DMA] * 7,
    )(flux2, tid, fid, etav)


def kernel(flux, tile_idx, fib_idx, eta):
    etav = jnp.full((16,), eta, jnp.float32)
    out2 = _sc_call(flux.T, tile_idx, fib_idx, etav)
    return out2.T


# speedup vs baseline: 26.6541x; 1.0023x over previous
import jax
import jax.numpy as jnp
from jax import lax
from jax.experimental import pallas as pl
from jax.experimental.pallas import tpu as pltpu
from jax.experimental.pallas import tpu_sc as plsc

N_TILES = 8
N_FIBRES = 5000
J = 20000
B = 128
BC = 32
NCHUNK = B // BC
ROWS_PAD = N_FIBRES + 2
A_ROWS = 40960
PER_TILE = 1280
IDXR = PER_TILE // 128
LAST = J - 15 * PER_TILE
WAVE = 4
DUMP_T = 8
DUMP_F = 483


def _body(flux2, tid, fid, etav, out2,
          A, B0, B1, B2, B3, B4, B5, B6, B7, Z,
          tI, fI, c0, cm, cp, ev,
          semL, semS0, semS1, semG0, semG1, semO0, semO1):
    cid = lax.axis_index("c")
    sid = lax.axis_index("s")
    Bufs = (B0, B1, B2, B3, B4, B5, B6, B7)
    semS = (semS0, semS1)
    semG = (semG0, semG1)
    semO = (semO0, semO1)
    Obuf = (B6, B7)

    pltpu.sync_copy(etav, ev)
    e = ev[...]
    cc = 1.0 - 2.0 * e
    cn = e

    jbase = sid * PER_TILE

    @pl.when(sid < 15)
    def _():
        pltpu.async_copy(tid.at[pl.ds(jbase, PER_TILE)], tI, semG0).wait()
        pltpu.async_copy(fid.at[pl.ds(jbase, PER_TILE)], fI, semG1).wait()

    @pl.when(sid == 15)
    def _():
        pltpu.async_copy(tid.at[pl.ds(15 * PER_TILE, LAST)],
                         tI.at[pl.ds(0, LAST)], semG0).wait()
        pltpu.async_copy(fid.at[pl.ds(15 * PER_TILE, LAST)],
                         fI.at[pl.ds(0, LAST)], semG1).wait()
        dt = jnp.full((16,), DUMP_T, jnp.int32)
        df = jnp.full((16,), DUMP_F, jnp.int32)
        for q in range(LAST, PER_TILE, 16):
            tI[pl.ds(q, 16)] = dt
            fI[pl.ds(q, 16)] = df

    def _compute_indices():
        for r in range(IDXR):
            for l in range(8):
                s = pl.ds(l * 16, 16)
                e0 = pl.ds(r * 128 + l * 16, 16)
                c = tI[e0] * ROWS_PAD + fI[e0] + 1
                c0[r, s] = c
                cm[r, s] = c - 1
                cp[r, s] = c + 1

    z16 = jnp.zeros((16,), jnp.float32)

    def _zb(i, carry):
        Z[i, pl.ds(0, 16)] = z16
        Z[i, pl.ds(16, 16)] = z16
        return carry

    def _fire_zero():
        return [pltpu.async_copy(
                    Z, A.at[pl.ds(sid * 2560 + z * 256, 256)], semL)
                for z in range(2560 // 256)]

    def _scatter_phase(pieces, co):
        waves = [pieces[i:i + WAVE] for i in range(0, len(pieces), WAVE)]

        def _load(s, rows, b):
            src = flux2.at[pl.ds(jbase + s * 128, rows), pl.ds(co, BC)]
            dst = Bufs[b] if rows == 128 else Bufs[b].at[pl.ds(0, rows)]
            return pltpu.async_copy(src, dst, semL)

        def _fire_loads(w):
            g = (w % 2) * WAVE
            return [_load(s, rows, g + i)
                    for i, (s, rows) in enumerate(waves[w])]

        dl = _fire_loads(0)
        dsc = {}
        for w in range(len(waves)):
            for d in dl:
                d.wait()
            g = (w % 2) * WAVE
            dsc[w] = [pltpu.async_copy(Bufs[g + i], A.at[c0.at[s]],
                                       semS[w % 2], add=True)
                      for i, (s, _) in enumerate(waves[w])]
            if w + 1 < len(waves):
                if w - 1 >= 0:
                    for d in dsc[w - 1]:
                        d.wait()
                dl = _fire_loads(w + 1)
        for w in range(max(0, len(waves) - 2), len(waves)):
            for d in dsc[w]:
                d.wait()

    def _comb(t, dst):
        g0r, g1r, g2r = Bufs[t * 3], Bufs[t * 3 + 1], Bufs[t * 3 + 2]
        O = Obuf[dst]

        def body(i, carry):
            r0 = i * 4
            for u in range(4):
                for c2 in (0, 16):
                    sl = pl.ds(c2, 16)
                    g0 = g0r[r0 + u, sl]
                    g12 = g1r[r0 + u, sl] + g2r[r0 + u, sl]
                    O[r0 + u, sl] = g0 * cc + g12 * cn
            return carry

        lax.fori_loop(0, 128 // 4, body, 0)

    def _gather_phase(pieces, co):
        npc = len(pieces)

        def _fire_g(s):
            t = (s % 2) * 3
            sg = semG[s % 2]
            return [pltpu.async_copy(A.at[c0.at[s]], Bufs[t], sg),
                    pltpu.async_copy(A.at[cm.at[s]], Bufs[t + 1], sg),
                    pltpu.async_copy(A.at[cp.at[s]], Bufs[t + 2], sg)]

        gd = {0: _fire_g(pieces[0][0])}
        if npc > 1:
            gd[1] = _fire_g(pieces[1][0])
        od = {}
        for i in range(npc):
            s, rows = pieces[i]
            for d in gd[i]:
                d.wait()
            if i >= 2:
                od[i - 2].wait()
            _comb(i % 2, i % 2)
            if i + 2 < npc:
                gd[i + 2] = _fire_g(pieces[i + 2][0])
            O = Obuf[i % 2]
            src = O if rows == 128 else O.at[pl.ds(0, rows)]
            od[i] = pltpu.async_copy(
                src, out2.at[pl.ds(jbase + s * 128, rows), pl.ds(co, BC)],
                semO[i % 2])
        for i in range(max(0, npc - 2), npc):
            od[i].wait()

    full = [(s, 128) for s in range(IDXR)]
    short = [(s, 128) for s in range(LAST // 128)] + [(LAST // 128, LAST % 128)]

    lax.fori_loop(0, 256, _zb, 0)
    zd = _fire_zero()
    _compute_indices()
    for d in zd:
        d.wait()
    plsc.subcore_barrier()

    def _chunk(k, carry):
        ci = cid * 2 + k
        co = pl.multiple_of(ci * BC, 8)

        with jax.named_scope("scat"):
            @pl.when(sid < 15)
            def _():
                _scatter_phase(full, co)

            @pl.when(sid == 15)
            def _():
                _scatter_phase(short, co)
            plsc.subcore_barrier()

        with jax.named_scope("gath"):
            @pl.when(sid < 15)
            def _():
                _gather_phase(full, co)

            @pl.when(sid == 15)
            def _():
                _gather_phase(short, co)
            plsc.subcore_barrier()

        @pl.when(k == 0)
        def _():
            zd2 = _fire_zero()
            for d in zd2:
                d.wait()
            plsc.subcore_barrier()
        return carry

    lax.fori_loop(0, 2, _chunk, 0)


def _sc_call(flux2, tid, fid, etav):
    mesh = plsc.VectorSubcoreMesh(core_axis_name="c", subcore_axis_name="s")
    return pl.kernel(
        _body,
        out_type=jax.ShapeDtypeStruct((J, B), jnp.float32),
        mesh=mesh,
        compiler_params=pltpu.CompilerParams(use_tc_tiling_on_sc=False),
        scratch_types=[
            pltpu.VMEM_SHARED((A_ROWS, BC), jnp.float32),
        ] + [pltpu.VMEM((128, BC), jnp.float32)] * 8 +
        [
            pltpu.VMEM((256, BC), jnp.float32),
            pltpu.VMEM((PER_TILE,), jnp.int32),
            pltpu.VMEM((PER_TILE,), jnp.int32),
            pltpu.VMEM((IDXR, 128), jnp.int32),
            pltpu.VMEM((IDXR, 128), jnp.int32),
            pltpu.VMEM((IDXR, 128), jnp.int32),
            pltpu.VMEM((16,), jnp.float32),
        ] + [pltpu.SemaphoreType.DMA] * 7,
    )(flux2, tid, fid, etav)


def kernel(flux, tile_idx, fib_idx, eta):
    etav = jnp.full((16,), eta, jnp.float32)
    out2 = _sc_call(flux.T, tile_idx, fib_idx, etav)
    return out2.T
